# baseline (speedup 1.0000x reference)
import functools

import jax
import jax.numpy as jnp
import numpy as np
from jax import lax
from jax.experimental import pallas as pl
from jax.experimental.pallas import tpu as pltpu
from jax.experimental.pallas import tpu_sc as plsc

MAX_LEN = 200
EMB_DIM = 64
BATCH = 4096

NUM_CORES = 2
NUM_SUBCORES = 16
NUM_WORKERS = NUM_CORES * NUM_SUBCORES
SEQ_PER_WORKER = BATCH // NUM_WORKERS
ROWS_PER_WORKER = SEQ_PER_WORKER * MAX_LEN


def _pe_const():
    pos = np.expand_dims(np.arange(MAX_LEN), 1)
    pe = pos / np.power(1000, 2 * np.expand_dims(np.arange(EMB_DIM) // 2, 0) / EMB_DIM)
    pe[:, 0::2] = np.sin(pe[:, 0::2])
    pe[:, 1::2] = np.cos(pe[:, 1::2])
    return pe.astype(np.float32)


_PE = _pe_const()


@functools.partial(
    pl.kernel,
    out_type=jax.ShapeDtypeStruct((BATCH * MAX_LEN, EMB_DIM), jnp.float32),
    mesh=plsc.VectorSubcoreMesh(core_axis_name="c", subcore_axis_name="s"),
    scratch_types=[
        pltpu.VMEM((ROWS_PER_WORKER,), jnp.int32),
        pltpu.VMEM((MAX_LEN, EMB_DIM), jnp.float32),
        pltpu.VMEM((MAX_LEN, EMB_DIM), jnp.float32),
        pltpu.SemaphoreType.DMA,
    ],
    compiler_params=pltpu.CompilerParams(use_tc_tiling_on_sc=False),
)
def _emb_lookup(idx_hbm, table_hbm, pe_hbm, out_hbm, idx_v, pe_v, buf, sem):
    wid = lax.axis_index("s") * NUM_CORES + lax.axis_index("c")
    base = wid * ROWS_PER_WORKER
    pltpu.sync_copy(idx_hbm.at[pl.ds(base, ROWS_PER_WORKER)], idx_v)
    pltpu.sync_copy(pe_hbm, pe_v)

    def seq_body(s, carry):
        off = s * MAX_LEN
        cp0 = pltpu.async_copy(
            table_hbm.at[idx_v.at[pl.ds(off, 128)]], buf.at[pl.ds(0, 128)], sem
        )
        cp1 = pltpu.async_copy(
            table_hbm.at[idx_v.at[pl.ds(off + 128, MAX_LEN - 128)]],
            buf.at[pl.ds(128, MAX_LEN - 128)],
            sem,
        )
        cp0.wait()
        cp1.wait()

        def pe_body(t, c):
            for d in range(EMB_DIM // 16):
                plsc.addupdate(
                    buf.at[t, pl.ds(d * 16, 16)], pe_v[t, pl.ds(d * 16, 16)]
                )
            return c

        lax.fori_loop(0, MAX_LEN, pe_body, 0)
        pltpu.sync_copy(buf, out_hbm.at[pl.ds(base + off, MAX_LEN)])
        return carry

    lax.fori_loop(0, SEQ_PER_WORKER, seq_body, 0)


def kernel(x, table):
    idx = x.reshape(-1).astype(jnp.int32)
    out = _emb_lookup(idx, table, jnp.asarray(_PE))
    return out.reshape(BATCH, MAX_LEN, EMB_DIM)


# speedup vs baseline: 1.1215x; 1.1215x over previous
import functools

import jax
import jax.numpy as jnp
import numpy as np
from jax import lax
from jax.experimental import pallas as pl
from jax.experimental.pallas import tpu as pltpu
from jax.experimental.pallas import tpu_sc as plsc

MAX_LEN = 200
EMB_DIM = 64
BATCH = 4096

NUM_CORES = 2
NUM_SUBCORES = 16
NUM_WORKERS = NUM_CORES * NUM_SUBCORES
SEQ_PER_WORKER = BATCH // NUM_WORKERS
ROWS_PER_WORKER = SEQ_PER_WORKER * MAX_LEN


def _pe_const():
    pos = np.expand_dims(np.arange(MAX_LEN), 1)
    pe = pos / np.power(1000, 2 * np.expand_dims(np.arange(EMB_DIM) // 2, 0) / EMB_DIM)
    pe[:, 0::2] = np.sin(pe[:, 0::2])
    pe[:, 1::2] = np.cos(pe[:, 1::2])
    return pe.astype(np.float32)


_PE = _pe_const()


NBUF = 4


@functools.partial(
    pl.kernel,
    out_type=jax.ShapeDtypeStruct((BATCH * MAX_LEN, EMB_DIM), jnp.float32),
    mesh=plsc.VectorSubcoreMesh(core_axis_name="c", subcore_axis_name="s"),
    scratch_types=[
        pltpu.VMEM((ROWS_PER_WORKER,), jnp.int32),
        pltpu.VMEM((MAX_LEN, EMB_DIM), jnp.float32),
        [pltpu.VMEM((MAX_LEN, EMB_DIM), jnp.float32) for _ in range(NBUF)],
        pltpu.SemaphoreType.DMA,
    ],
    compiler_params=pltpu.CompilerParams(use_tc_tiling_on_sc=False),
)
def _emb_lookup(idx_hbm, table_hbm, pe_hbm, out_hbm, idx_v, pe_v, bufs, gsem):
    wid = lax.axis_index("s") * NUM_CORES + lax.axis_index("c")
    base = wid * ROWS_PER_WORKER
    pltpu.sync_copy(idx_hbm.at[pl.ds(base, ROWS_PER_WORKER)], idx_v)
    pltpu.sync_copy(pe_hbm, pe_v)

    def gather_start(s, buf):
        off = s * MAX_LEN
        pltpu.async_copy(
            table_hbm.at[idx_v.at[pl.ds(off, 128)]], buf.at[pl.ds(0, 128)], gsem
        )
        pltpu.async_copy(
            table_hbm.at[idx_v.at[pl.ds(off + 128, MAX_LEN - 128)]],
            buf.at[pl.ds(128, MAX_LEN - 128)],
            gsem,
        )

    def gather_wait(buf):
        pltpu.make_async_copy(
            table_hbm.at[idx_v.at[pl.ds(0, 128)]], buf.at[pl.ds(0, 128)], gsem
        ).wait()
        pltpu.make_async_copy(
            table_hbm.at[idx_v.at[pl.ds(0, MAX_LEN - 128)]],
            buf.at[pl.ds(128, MAX_LEN - 128)],
            gsem,
        ).wait()

    def pe_add(buf):
        def body(t8, c):
            for u in range(8):
                t = t8 * 8 + u
                for d in range(EMB_DIM // 16):
                    plsc.addupdate(
                        buf.at[t, pl.ds(d * 16, 16)], pe_v[t, pl.ds(d * 16, 16)]
                    )
            return c

        lax.fori_loop(0, MAX_LEN // 8, body, 0)

    for j in range(NBUF - 1):
        gather_start(j, bufs[j])

    def group_body(k, carry):
        for j in range(NBUF):
            s = k * NBUF + j
            @pl.when(s + NBUF - 1 < SEQ_PER_WORKER)
            def _():
                gather_start(s + NBUF - 1, bufs[(j + NBUF - 1) % NBUF])

            gather_wait(bufs[j])
            pe_add(bufs[j])
            pltpu.sync_copy(
                bufs[j], out_hbm.at[pl.ds(base + s * MAX_LEN, MAX_LEN)]
            )
        return carry

    lax.fori_loop(0, SEQ_PER_WORKER // NBUF, group_body, 0)


def kernel(x, table):
    idx = x.reshape(-1).astype(jnp.int32)
    out = _emb_lookup(idx, table, jnp.asarray(_PE))
    return out.reshape(BATCH, MAX_LEN, EMB_DIM)


# speedup vs baseline: 1.1687x; 1.0421x over previous
import functools

import jax
import jax.numpy as jnp
import numpy as np
from jax import lax
from jax.experimental import pallas as pl
from jax.experimental.pallas import tpu as pltpu
from jax.experimental.pallas import tpu_sc as plsc

MAX_LEN = 200
EMB_DIM = 64
BATCH = 4096

NUM_CORES = 2
NUM_SUBCORES = 16
NUM_WORKERS = NUM_CORES * NUM_SUBCORES
BLOCK = 512
NBLK = BATCH // BLOCK


def _pe_const():
    pos = np.expand_dims(np.arange(MAX_LEN), 1)
    pe = pos / np.power(1000, 2 * np.expand_dims(np.arange(EMB_DIM) // 2, 0) / EMB_DIM)
    pe[:, 0::2] = np.sin(pe[:, 0::2])
    pe[:, 1::2] = np.cos(pe[:, 1::2])
    return pe.astype(np.float32)


_PE = _pe_const()


@functools.partial(
    pl.kernel,
    out_type=jax.ShapeDtypeStruct((MAX_LEN, BATCH, EMB_DIM), jnp.float32),
    mesh=plsc.VectorSubcoreMesh(core_axis_name="c", subcore_axis_name="s"),
    scratch_types=[
        pltpu.VMEM((BATCH,), jnp.int32),
        pltpu.VMEM((MAX_LEN, EMB_DIM), jnp.float32),
        [pltpu.VMEM((BLOCK, EMB_DIM), jnp.float32) for _ in range(2)],
        pltpu.SemaphoreType.DMA,
        pltpu.SemaphoreType.DMA,
    ],
    compiler_params=pltpu.CompilerParams(use_tc_tiling_on_sc=False),
)
def _emb_lookup(xt_hbm, table_hbm, pe_hbm, out_hbm, idx_v, pe_v, bufs, gsem, osem):
    wid = lax.axis_index("s") * NUM_CORES + lax.axis_index("c")
    t_lo = wid * MAX_LEN // NUM_WORKERS
    t_hi = (wid + 1) * MAX_LEN // NUM_WORKERS
    pltpu.sync_copy(pe_hbm, pe_v)

    def gather_start(bi, buf):
        for j in range(BLOCK // 128):
            pltpu.async_copy(
                table_hbm.at[idx_v.at[pl.ds(bi * BLOCK + j * 128, 128)]],
                buf.at[pl.ds(j * 128, 128)],
                gsem,
            )

    def gather_wait(buf):
        for j in range(BLOCK // 128):
            pltpu.make_async_copy(
                table_hbm.at[idx_v.at[pl.ds(0, 128)]],
                buf.at[pl.ds(j * 128, 128)],
                gsem,
            ).wait()

    def pe_add(buf, t):
        pe_regs = [pe_v[t, pl.ds(16 * j, 16)] for j in range(EMB_DIM // 16)]

        def body(r8, c):
            for u in range(8):
                r = r8 * 8 + u
                for j in range(EMB_DIM // 16):
                    plsc.addupdate(buf.at[r, pl.ds(16 * j, 16)], pe_regs[j])
            return c

        lax.fori_loop(0, BLOCK // 8, body, 0)

    def out_start(bi, buf, t):
        pltpu.async_copy(
            buf, out_hbm.at[t, pl.ds(bi * BLOCK, BLOCK)], osem
        )

    def out_wait(buf, t):
        pltpu.make_async_copy(
            buf, out_hbm.at[t, pl.ds(0, BLOCK)], osem
        ).wait()

    def t_body(t, carry):
        pltpu.sync_copy(xt_hbm.at[t], idx_v)
        gather_start(0, bufs[0])
        for bi in range(NBLK):
            cur, nxt = bufs[bi % 2], bufs[(bi + 1) % 2]
            if bi + 1 < NBLK:
                if bi >= 1:
                    out_wait(nxt, t)
                gather_start(bi + 1, nxt)
            gather_wait(cur)
            pe_add(cur, t)
            out_start(bi, cur, t)
        out_wait(bufs[(NBLK - 2) % 2], t)
        out_wait(bufs[(NBLK - 1) % 2], t)
        return carry

    lax.fori_loop(t_lo, t_hi, t_body, 0)


def kernel(x, table):
    xt = jnp.transpose(x).astype(jnp.int32)
    out3 = _emb_lookup(xt, table, jnp.asarray(_PE))
    return jnp.transpose(out3, (1, 0, 2))
